# Initial kernel scaffold; baseline (speedup 1.0000x reference)
import jax
import jax.numpy as jnp
from jax.experimental import pallas as pl


def kernel(row_idx, col_idx, data_tensor, num_ids, embeddings):
    raise NotImplementedError("write your pallas kernel here")


# speedup vs baseline: 1.8050x; 1.8050x over previous
import jax
import jax.numpy as jnp
from jax import lax
from jax.experimental import pallas as pl
from jax.experimental.pallas import tpu as pltpu
from jax.experimental.pallas import tpu_sc as plsc

NNZ = 819200
NUM_IDS = 16384
EMBED_DIM = 64
NUM_CORES = 2
NUM_SUBCORES = 16
NUM_WORKERS = NUM_CORES * NUM_SUBCORES
CHUNK = NNZ // NUM_WORKERS
K = 128
STEPS = CHUNK // K
ROWS_PER_TILE = NUM_IDS // NUM_SUBCORES


def _sc_body(row_hbm, col_hbm, data_hbm, emb_hbm, zeros_hbm, partial_hbm,
             col_v, row_v, data_v, rows_v, acc, sem):
    c = lax.axis_index("c")
    s = lax.axis_index("s")
    w = c * NUM_SUBCORES + s

    pltpu.sync_copy(zeros_hbm.at[pl.ds(s * ROWS_PER_TILE, ROWS_PER_TILE)],
                    acc.at[pl.ds(s * ROWS_PER_TILE, ROWS_PER_TILE)])
    plsc.subcore_barrier()

    base0 = w * CHUNK

    def step(t, carry):
        base = base0 + t * K
        pltpu.sync_copy(col_hbm.at[pl.ds(base, K)], col_v)
        pltpu.sync_copy(row_hbm.at[pl.ds(base, K)], row_v)
        pltpu.sync_copy(data_hbm.at[pl.ds(base, K)], data_v)
        pltpu.async_copy(emb_hbm.at[col_v], rows_v, sem).wait()

        def scale(j, acc_c):
            dv = data_v[pl.ds(j * 16, 16)]
            for l in range(16):
                wv = dv.at[jnp.full((16,), l, dtype=jnp.int32)].get(
                    mode="promise_in_bounds")
                i = j * 16 + l
                for q in range(EMBED_DIM // 16):
                    sl = pl.ds(q * 16, 16)
                    rows_v[i, sl] = rows_v[i, sl] * wv
            return acc_c

        lax.fori_loop(0, K // 16, scale, 0)
        pltpu.sync_copy(rows_v, acc.at[row_v], add=True)
        return carry

    lax.fori_loop(0, STEPS, step, 0)
    plsc.subcore_barrier()

    dst = c * NUM_IDS + s * ROWS_PER_TILE
    pltpu.sync_copy(acc.at[pl.ds(s * ROWS_PER_TILE, ROWS_PER_TILE)],
                    partial_hbm.at[pl.ds(dst, ROWS_PER_TILE)])


_sc_call = pl.kernel(
    _sc_body,
    out_type=jax.ShapeDtypeStruct((NUM_CORES * NUM_IDS, EMBED_DIM),
                                  jnp.float32),
    mesh=plsc.VectorSubcoreMesh(core_axis_name="c", subcore_axis_name="s",
                                num_cores=NUM_CORES,
                                num_subcores=NUM_SUBCORES),
    scratch_types=[
        pltpu.VMEM((K,), jnp.int32),
        pltpu.VMEM((K,), jnp.int32),
        pltpu.VMEM((K,), jnp.float32),
        pltpu.VMEM((K, EMBED_DIM), jnp.float32),
        pltpu.VMEM_SHARED((NUM_IDS, EMBED_DIM), jnp.float32),
        pltpu.SemaphoreType.DMA,
    ],
    compiler_params=pltpu.CompilerParams(use_tc_tiling_on_sc=False),
)

_BLK = 1024


def _combine_body(p0_ref, p1_ref, o_ref):
    o_ref[...] = p0_ref[...] + p1_ref[...]


def _combine(partial):
    n_blk = NUM_IDS // _BLK
    return pl.pallas_call(
        _combine_body,
        out_shape=jax.ShapeDtypeStruct((NUM_IDS, EMBED_DIM), jnp.float32),
        grid=(n_blk,),
        in_specs=[
            pl.BlockSpec((_BLK, EMBED_DIM), lambda i: (i, 0)),
            pl.BlockSpec((_BLK, EMBED_DIM), lambda i, n=n_blk: (i + n, 0)),
        ],
        out_specs=pl.BlockSpec((_BLK, EMBED_DIM), lambda i: (i, 0)),
    )(partial, partial)


def kernel(row_idx, col_idx, data_tensor, num_ids, embeddings):
    del num_ids
    row_idx = row_idx.astype(jnp.int32)
    col_idx = col_idx.astype(jnp.int32)
    zeros = jnp.zeros((NUM_IDS, EMBED_DIM), jnp.float32)
    partial = _sc_call(row_idx, col_idx, data_tensor, embeddings, zeros)
    return _combine(partial)


# speedup vs baseline: 2.3254x; 1.2883x over previous
import jax
import jax.numpy as jnp
from jax import lax
from jax.experimental import pallas as pl
from jax.experimental.pallas import tpu as pltpu
from jax.experimental.pallas import tpu_sc as plsc

NNZ = 819200
NUM_IDS = 16384
EMBED_DIM = 64
NUM_CORES = 2
NUM_SUBCORES = 16
NUM_WORKERS = NUM_CORES * NUM_SUBCORES
CHUNK = NNZ // NUM_WORKERS
K = 128
STEPS = CHUNK // K
ROWS_PER_TILE = NUM_IDS // NUM_SUBCORES


def _sc_body(row_hbm, col_hbm, data_hbm, emb_hbm, zeros_hbm, partial_hbm,
             col0, col1, row0, row1, dat0, dat1, rows0, rows1, acc,
             sem_i0, sem_i1, sem_g0, sem_g1):
    cols = (col0, col1)
    rowids = (row0, row1)
    dats = (dat0, dat1)
    bufs = (rows0, rows1)
    sem_i = (sem_i0, sem_i1)
    sem_g = (sem_g0, sem_g1)

    c = lax.axis_index("c")
    s = lax.axis_index("s")
    w = c * NUM_SUBCORES + s

    pltpu.sync_copy(zeros_hbm.at[pl.ds(s * ROWS_PER_TILE, ROWS_PER_TILE)],
                    acc.at[pl.ds(s * ROWS_PER_TILE, ROWS_PER_TILE)])
    plsc.subcore_barrier()

    base0 = w * CHUNK

    def issue_idx(t, b):
        base = base0 + t * K
        pltpu.async_copy(col_hbm.at[pl.ds(base, K)], cols[b], sem_i[b])
        pltpu.async_copy(row_hbm.at[pl.ds(base, K)], rowids[b], sem_i[b])
        pltpu.async_copy(data_hbm.at[pl.ds(base, K)], dats[b], sem_i[b])

    def wait_idx(t, b):
        base = base0 + t * K
        pltpu.make_async_copy(col_hbm.at[pl.ds(base, K)], cols[b],
                              sem_i[b]).wait()
        pltpu.make_async_copy(row_hbm.at[pl.ds(base, K)], rowids[b],
                              sem_i[b]).wait()
        pltpu.make_async_copy(data_hbm.at[pl.ds(base, K)], dats[b],
                              sem_i[b]).wait()

    def issue_gather(b):
        pltpu.async_copy(emb_hbm.at[cols[b]], bufs[b], sem_g[b])

    def wait_gather(b):
        pltpu.make_async_copy(emb_hbm.at[cols[b]], bufs[b], sem_g[b]).wait()

    issue_idx(0, 0)
    issue_idx(1, 1)
    wait_idx(0, 0)
    issue_gather(0)

    def outer(t2, carry):
        for b in range(2):
            t = 2 * t2 + b
            wait_gather(b)

            @pl.when(t + 1 < STEPS)
            def _():
                wait_idx(t + 1, 1 - b)
                issue_gather(1 - b)

            def scale(j, acc_c):
                dv = dats[b][pl.ds(j * 16, 16)]
                for l in range(16):
                    wv = dv.at[jnp.full((16,), l, dtype=jnp.int32)].get(
                        mode="promise_in_bounds")
                    i = j * 16 + l
                    for q in range(EMBED_DIM // 16):
                        sl = pl.ds(q * 16, 16)
                        bufs[b][i, sl] = bufs[b][i, sl] * wv
                return acc_c

            lax.fori_loop(0, K // 16, scale, 0)

            pltpu.sync_copy(bufs[b], acc.at[rowids[b]], add=True)

            @pl.when(t + 2 < STEPS)
            def _():
                issue_idx(t + 2, b)
        return carry

    lax.fori_loop(0, STEPS // 2, outer, 0)
    plsc.subcore_barrier()

    dst = c * NUM_IDS + s * ROWS_PER_TILE
    pltpu.sync_copy(acc.at[pl.ds(s * ROWS_PER_TILE, ROWS_PER_TILE)],
                    partial_hbm.at[pl.ds(dst, ROWS_PER_TILE)])


_sc_call = pl.kernel(
    _sc_body,
    out_type=jax.ShapeDtypeStruct((NUM_CORES * NUM_IDS, EMBED_DIM),
                                  jnp.float32),
    mesh=plsc.VectorSubcoreMesh(core_axis_name="c", subcore_axis_name="s",
                                num_cores=NUM_CORES,
                                num_subcores=NUM_SUBCORES),
    scratch_types=[
        pltpu.VMEM((K,), jnp.int32),
        pltpu.VMEM((K,), jnp.int32),
        pltpu.VMEM((K,), jnp.int32),
        pltpu.VMEM((K,), jnp.int32),
        pltpu.VMEM((K,), jnp.float32),
        pltpu.VMEM((K,), jnp.float32),
        pltpu.VMEM((K, EMBED_DIM), jnp.float32),
        pltpu.VMEM((K, EMBED_DIM), jnp.float32),
        pltpu.VMEM_SHARED((NUM_IDS, EMBED_DIM), jnp.float32),
        pltpu.SemaphoreType.DMA,
        pltpu.SemaphoreType.DMA,
        pltpu.SemaphoreType.DMA,
        pltpu.SemaphoreType.DMA,
    ],
    compiler_params=pltpu.CompilerParams(use_tc_tiling_on_sc=False),
)

_BLK = 1024


def _combine_body(p0_ref, p1_ref, o_ref):
    o_ref[...] = p0_ref[...] + p1_ref[...]


def _combine(partial):
    n_blk = NUM_IDS // _BLK
    return pl.pallas_call(
        _combine_body,
        out_shape=jax.ShapeDtypeStruct((NUM_IDS, EMBED_DIM), jnp.float32),
        grid=(n_blk,),
        in_specs=[
            pl.BlockSpec((_BLK, EMBED_DIM), lambda i: (i, 0)),
            pl.BlockSpec((_BLK, EMBED_DIM), lambda i, n=n_blk: (i + n, 0)),
        ],
        out_specs=pl.BlockSpec((_BLK, EMBED_DIM), lambda i: (i, 0)),
    )(partial, partial)


def kernel(row_idx, col_idx, data_tensor, num_ids, embeddings):
    del num_ids
    row_idx = row_idx.astype(jnp.int32)
    col_idx = col_idx.astype(jnp.int32)
    zeros = jnp.zeros((NUM_IDS, EMBED_DIM), jnp.float32)
    partial = _sc_call(row_idx, col_idx, data_tensor, embeddings, zeros)
    return _combine(partial)
